# baseline (speedup 1.0000x reference)
import functools

import jax
import jax.numpy as jnp
from jax import lax
from jax.experimental import pallas as pl
from jax.experimental.pallas import tpu as pltpu
from jax.experimental.pallas import tpu_sc as plsc

D = 64
L_SEQ = 200
NUM_CORES = 2
NUM_SUBCORES = 16
NUM_WORKERS = NUM_CORES * NUM_SUBCORES
LANES = 16

SEQS_PER_CHUNK = 4
CHUNK_ROWS = SEQS_PER_CHUNK * L_SEQ


@functools.lru_cache(maxsize=None)
def _build_call(n_rows: int):
    rows_per_w = n_rows // NUM_WORKERS
    n_chunks = rows_per_w // CHUNK_ROWS
    assert rows_per_w % CHUNK_ROWS == 0

    mesh = plsc.VectorSubcoreMesh(core_axis_name="c", subcore_axis_name="s")

    @functools.partial(
        pl.kernel,
        mesh=mesh,
        out_type=jax.ShapeDtypeStruct((n_rows, D), jnp.float32),
        compiler_params=pltpu.CompilerParams(use_tc_tiling_on_sc=False),
        scratch_types=[
            pltpu.VMEM((rows_per_w,), jnp.int32),
            pltpu.VMEM((CHUNK_ROWS, D), jnp.float32),
            pltpu.VMEM((L_SEQ, D), jnp.float32),
            pltpu.SemaphoreType.DMA,
        ],
    )
    def emb(ids_hbm, table_hbm, pos_hbm, out_hbm, idx_v, rows_v, pos_v, sem):
        wid = lax.axis_index("s") * NUM_CORES + lax.axis_index("c")
        base = wid * rows_per_w
        pltpu.sync_copy(pos_hbm, pos_v)
        pltpu.sync_copy(ids_hbm.at[pl.ds(base, rows_per_w)], idx_v)

        for ck in range(n_chunks):
            pltpu.async_copy(
                table_hbm.at[idx_v.at[pl.ds(ck * CHUNK_ROWS, CHUNK_ROWS)]],
                rows_v,
                sem,
            ).wait()

            def seq_body(s, _):
                def row_body(l, _):
                    r = s * L_SEQ + l
                    for c in range(D // LANES):
                        sl = pl.ds(c * LANES, LANES)
                        rows_v[r, sl] = rows_v[r, sl] + pos_v[l, sl]
                    return 0

                lax.fori_loop(0, L_SEQ, row_body, 0)
                return 0

            lax.fori_loop(0, SEQS_PER_CHUNK, seq_body, 0)

            pltpu.sync_copy(
                rows_v, out_hbm.at[pl.ds(base + ck * CHUNK_ROWS, CHUNK_ROWS)]
            )

    return emb


def kernel(input_ids, token_table, position_table):
    b, l = input_ids.shape
    ids_flat = input_ids.reshape(b * l).astype(jnp.int32)
    pos = position_table[:l]
    out = _build_call(b * l)(ids_flat, token_table, pos)
    return out.reshape(b, l, D)


# speedup vs baseline: 1.0439x; 1.0439x over previous
import functools

import jax
import jax.numpy as jnp
from jax import lax
from jax.experimental import pallas as pl
from jax.experimental.pallas import tpu as pltpu
from jax.experimental.pallas import tpu_sc as plsc

D = 64
L_SEQ = 200
NUM_CORES = 2
NUM_SUBCORES = 16
NUM_WORKERS = NUM_CORES * NUM_SUBCORES
LANES = 16

SEQS_PER_CHUNK = 4
CHUNK_ROWS = SEQS_PER_CHUNK * L_SEQ


@functools.lru_cache(maxsize=None)
def _build_call(n_rows: int):
    rows_per_w = n_rows // NUM_WORKERS
    n_chunks = rows_per_w // CHUNK_ROWS
    assert rows_per_w % CHUNK_ROWS == 0

    mesh = plsc.VectorSubcoreMesh(core_axis_name="c", subcore_axis_name="s")

    @functools.partial(
        pl.kernel,
        mesh=mesh,
        out_type=jax.ShapeDtypeStruct((n_rows, D), jnp.float32),
        compiler_params=pltpu.CompilerParams(use_tc_tiling_on_sc=False),
        scratch_types=[
            pltpu.VMEM((rows_per_w,), jnp.int32),
            pltpu.VMEM((CHUNK_ROWS, D), jnp.float32),
            pltpu.VMEM((CHUNK_ROWS, D), jnp.float32),
            pltpu.VMEM((L_SEQ, D), jnp.float32),
            pltpu.SemaphoreType.DMA,
            pltpu.SemaphoreType.DMA,
        ],
    )
    def emb(ids_hbm, table_hbm, pos_hbm, out_hbm, idx_v, buf0, buf1, pos_v,
            gsem, osem):
        bufs = (buf0, buf1)
        wid = lax.axis_index("s") * NUM_CORES + lax.axis_index("c")
        base = wid * rows_per_w
        pltpu.sync_copy(pos_hbm, pos_v)
        pltpu.sync_copy(ids_hbm.at[pl.ds(base, rows_per_w)], idx_v)

        def start_gather(ck):
            return pltpu.async_copy(
                table_hbm.at[idx_v.at[pl.ds(ck * CHUNK_ROWS, CHUNK_ROWS)]],
                bufs[ck % 2],
                gsem,
            )

        gather_desc = start_gather(0)
        scatter_descs = [None, None]
        for ck in range(n_chunks):
            cur = bufs[ck % 2]
            gather_desc.wait()
            if ck + 1 < n_chunks:
                if scatter_descs[(ck + 1) % 2] is not None:
                    scatter_descs[(ck + 1) % 2].wait()
                    scatter_descs[(ck + 1) % 2] = None
                gather_desc = start_gather(ck + 1)

            @plsc.parallel_loop(0, L_SEQ)
            def _(l):
                pv = [pos_v[l, pl.ds(c * LANES, LANES)] for c in range(D // LANES)]
                for s in range(SEQS_PER_CHUNK):
                    r = s * L_SEQ + l
                    for c in range(D // LANES):
                        sl = pl.ds(c * LANES, LANES)
                        cur[r, sl] = cur[r, sl] + pv[c]

            scatter_descs[ck % 2] = pltpu.async_copy(
                cur, out_hbm.at[pl.ds(base + ck * CHUNK_ROWS, CHUNK_ROWS)], osem
            )
        for d in scatter_descs:
            if d is not None:
                d.wait()

    return emb


def kernel(input_ids, token_table, position_table):
    b, l = input_ids.shape
    ids_flat = input_ids.reshape(b * l).astype(jnp.int32)
    pos = position_table[:l]
    out = _build_call(b * l)(ids_flat, token_table, pos)
    return out.reshape(b, l, D)
